# Initial kernel scaffold; baseline (speedup 1.0000x reference)
import jax
import jax.numpy as jnp
from jax.experimental import pallas as pl


def kernel(x, pos_table):
    raise NotImplementedError("write your pallas kernel here")


# speedup vs baseline: 1.4473x; 1.4473x over previous
import jax
import jax.numpy as jnp
from jax.experimental import pallas as pl

MAXLEN = 8192
EMBED_DIM = 768
BATCH = 4
SEQ_BLK = 512


def _add_body(x_ref, pos_ref, out_ref):
    out_ref[0] = x_ref[0] + pos_ref[...]


def kernel(x, pos_table):
    grid = (MAXLEN // SEQ_BLK, BATCH)
    return pl.pallas_call(
        _add_body,
        grid=grid,
        in_specs=[
            pl.BlockSpec((1, SEQ_BLK, EMBED_DIM), lambda s, b: (b, s, 0)),
            pl.BlockSpec((SEQ_BLK, EMBED_DIM), lambda s, b: (s, 0)),
        ],
        out_specs=pl.BlockSpec((1, SEQ_BLK, EMBED_DIM), lambda s, b: (b, s, 0)),
        out_shape=jax.ShapeDtypeStruct((BATCH, MAXLEN, EMBED_DIM), jnp.float32),
    )(x, pos_table)
